# baseline (speedup 1.0000x reference)
import functools

import jax
import jax.numpy as jnp
from jax import lax
from jax.experimental import pallas as pl
from jax.experimental.pallas import tpu as pltpu
from jax.experimental.pallas import tpu_sc as plsc

B, J, K, H, V = 16, 4, 2048, 256, 65536
PAIRS = B * J
L = 16
NC, NS = 2, 16
NW = NC * NS
PPW = PAIRS // NW
VBLK = 4096
VS = 12 * VBLK
CH = 128
MAXC = K // CH


def _tc_scores(stims, embed):
    def mm(stims_ref, emb_ref, out_ref):
        out_ref[...] = lax.dot_general(
            stims_ref[...], emb_ref[...],
            dimension_numbers=(((1,), (1,)), ((), ())),
            preferred_element_type=jnp.float32,
            precision=lax.Precision.HIGHEST,
        )

    return pl.pallas_call(
        mm,
        grid=(VS // VBLK,),
        in_specs=[
            pl.BlockSpec((B, H), lambda i: (0, 0)),
            pl.BlockSpec((VBLK, H), lambda i: (i, 0)),
        ],
        out_specs=pl.BlockSpec((B, VBLK), lambda i: (0, i)),
        out_shape=jax.ShapeDtypeStruct((B, VS), jnp.float32),
    )(stims, embed)


def _sc_rowdot(embed, atn_idx, stims):
    mesh = plsc.VectorSubcoreMesh(core_axis_name="c", subcore_axis_name="s")

    @functools.partial(
        pl.kernel,
        mesh=mesh,
        compiler_params=pltpu.CompilerParams(needs_layout_passes=False),
        out_type=[jax.ShapeDtypeStruct((B, J, K), jnp.float32)],
        scratch_types=[
            pltpu.VMEM((PPW, K), jnp.int32),
            pltpu.VMEM((H,), jnp.float32),
            pltpu.VMEM((MAXC, CH), jnp.int32),
            pltpu.VMEM((K,), jnp.int32),
            pltpu.VMEM((PPW, K), jnp.float32),
            pltpu.VMEM((H,), jnp.float32),
            pltpu.VMEM((CH, H), jnp.float32),
            pltpu.VMEM((CH, H), jnp.float32),
            pltpu.SemaphoreType.DMA,
            pltpu.SemaphoreType.DMA,
            pltpu.SemaphoreType.DMA,
            pltpu.SemaphoreType.DMA,
        ],
    )
    def k(embed_hbm, idx_hbm, stims_hbm, x_hbm,
          idx_v, stims_v, cidx_v, cpos_v, xbuf_v, t_v, buf0, buf1,
          sem_in, semg0, semg1, sem_out):
        wid = lax.axis_index("s") * NC + lax.axis_index("c")
        b = wid // (NW // B)
        stage = [pltpu.async_copy(stims_hbm.at[b], stims_v, sem_in)]
        for jj in range(PPW):
            j = (wid * PPW + jj) % J
            stage.append(
                pltpu.async_copy(idx_hbm.at[b, j], idx_v.at[jj], sem_in))
        for c in stage:
            c.wait()

        s_list = [stims_v[pl.ds(h * L, L)] for h in range(H // L)]
        lane = lax.broadcasted_iota(jnp.int32, (L,), 0)
        zrow_i = jnp.zeros((L,), jnp.int32)

        def chunk_dot(cbase, buf, offv, jj):
            def group(g, carry):
                for r in range(L):
                    ridx = g * L + r
                    acc = buf[ridx, pl.ds(0, L)] * s_list[0]
                    for h in range(1, H // L):
                        acc = acc + buf[ridx, pl.ds(h * L, L)] * s_list[h]
                    t_v[pl.ds(r * L, L)] = acc
                sums = plsc.load_gather(t_v, [lane * L])
                for l in range(1, L):
                    sums = sums + plsc.load_gather(t_v, [lane * L + l])
                ent = cbase * CH + g * L
                pos_v = cpos_v[pl.ds(ent, L)]
                valid = (ent + lane) < offv
                plsc.store_scatter(
                    xbuf_v, [jnp.full((L,), jj, jnp.int32), pos_v],
                    sums, mask=valid)
                return carry
            lax.fori_loop(0, CH // L, group, 0)

        def do_pair(jj, _):
            def zbody(i, c):
                cidx_v[i, pl.ds(0, L)] = zrow_i
                for q in range(1, CH // L):
                    cidx_v[i, pl.ds(q * L, L)] = zrow_i
                return c
            lax.fori_loop(0, MAXC, zbody, 0)

            def cbody(i, offv):
                idxv = idx_v[jj, pl.ds(i * L, L)]
                m = idxv >= VS
                pref = plsc.cumsum(m.astype(jnp.int32))
                pos = offv + pref - 1
                cnt = plsc.all_reduce_population_count(m)
                plsc.store_scatter(
                    cidx_v,
                    [jnp.right_shift(pos, 7), pos & (CH - 1)],
                    idxv, mask=m)
                plsc.store_scatter(cpos_v, [pos], lane + i * L, mask=m)
                return offv + cnt
            offv = lax.fori_loop(0, K // L, cbody,
                                 jnp.zeros((L,), jnp.int32))
            tot = jnp.max(offv, axis=0)
            nc = (tot + CH - 1) // CH
            nce = ((nc + 1) // 2) * 2

            @pl.when(nce > 0)
            def _():
                pltpu.async_copy(embed_hbm.at[cidx_v.at[0]], buf0, semg0)

            @pl.when(nce > 1)
            def _():
                pltpu.async_copy(embed_hbm.at[cidx_v.at[1]], buf1, semg1)

            def mbody(i, carry):
                c0 = 2 * i
                pltpu.make_async_copy(
                    embed_hbm.at[cidx_v.at[0]], buf0, semg0).wait()
                chunk_dot(c0, buf0, offv, jj)

                @pl.when(c0 + 2 < nce)
                def _():
                    pltpu.async_copy(
                        embed_hbm.at[cidx_v.at[c0 + 2]], buf0, semg0)

                pltpu.make_async_copy(
                    embed_hbm.at[cidx_v.at[1]], buf1, semg1).wait()
                chunk_dot(c0 + 1, buf1, offv, jj)

                @pl.when(c0 + 3 < nce)
                def _():
                    pltpu.async_copy(
                        embed_hbm.at[cidx_v.at[c0 + 3]], buf1, semg1)
                return carry
            lax.fori_loop(0, nce // 2, mbody, 0)
            return _

        for jj in range(PPW):
            do_pair(jj, None)
            j = (wid * PPW + jj) % J
            pltpu.async_copy(xbuf_v.at[jj], x_hbm.at[b, j], sem_out)
        for jj in range(PPW):
            j = (wid * PPW + jj) % J
            pltpu.make_async_copy(
                xbuf_v.at[jj], x_hbm.at[b, j], sem_out).wait()

    (x_part,) = k(embed, atn_idx, stims)
    return x_part


def _sc_merge(scores, atn_idx, lens, prev_x):
    mesh = plsc.VectorSubcoreMesh(core_axis_name="c", subcore_axis_name="s")

    @functools.partial(
        pl.kernel,
        mesh=mesh,
        compiler_params=pltpu.CompilerParams(needs_layout_passes=False),
        out_type=[
            jax.ShapeDtypeStruct((B, J, K), jnp.float32),
            jax.ShapeDtypeStruct((PAIRS, L), jnp.int32),
        ],
        scratch_types=[
            pltpu.VMEM((VS,), jnp.float32),
            pltpu.VMEM((PPW, K), jnp.int32),
            pltpu.VMEM((PPW, K), jnp.float32),
            pltpu.VMEM((PPW, K), jnp.float32),
            pltpu.SemaphoreType.DMA,
            pltpu.SemaphoreType.DMA,
            pltpu.VMEM((PPW, L), jnp.int32),
            pltpu.VMEM((PAIRS,), jnp.int32),
        ],
    )
    def k(scores_hbm, idx_hbm, prev_hbm, lens_hbm, x_hbm, xidx_hbm,
          row_v, idx_v, xbuf_v, prev_v, sem_in, sem_out, xidx_v, lens_v):
        wid = lax.axis_index("s") * NC + lax.axis_index("c")
        b = wid // (NW // B)
        copies = [pltpu.async_copy(scores_hbm.at[b], row_v, sem_in),
                  pltpu.async_copy(lens_hbm, lens_v, sem_in)]
        for jj in range(PPW):
            j = (wid * PPW + jj) % J
            copies.append(
                pltpu.async_copy(idx_hbm.at[b, j], idx_v.at[jj], sem_in))
            copies.append(
                pltpu.async_copy(prev_hbm.at[b, j], prev_v.at[jj], sem_in))
        for c in copies:
            c.wait()
        lane = lax.broadcasted_iota(jnp.int32, (L,), 0)
        neg = jnp.full((L,), -1e9, jnp.float32)
        zero = jnp.zeros((L,), jnp.int32)
        writes = []
        for jj in range(PPW):
            p = wid * PPW + jj
            j = p % J
            ln = plsc.load_gather(lens_v, [jnp.full((L,), p, jnp.int32)])

            def body(i, carry, jj=jj, ln=ln):
                best_val, best_idx = carry
                idxv = idx_v[jj, pl.ds(i * L, L)]
                inrange = idxv < VS
                safe = jnp.where(inrange, idxv, zero)
                vals = plsc.load_gather(row_v, [safe])
                prev = prev_v[jj, pl.ds(i * L, L)]
                merged = jnp.where(inrange, vals, prev)
                xbuf_v[jj, pl.ds(i * L, L)] = merged
                kv = lane + i * L
                mval = jnp.where(kv < ln, merged, neg)
                upd = mval > best_val
                return (jnp.where(upd, mval, best_val),
                        jnp.where(upd, kv, best_idx))

            bv0 = jnp.full((L,), -jnp.inf, jnp.float32)
            bi0 = jnp.zeros((L,), jnp.int32)
            bv, bi = lax.fori_loop(0, K // L, body, (bv0, bi0))
            mx = jnp.max(bv, axis=0)
            cand = jnp.where(bv == mx, bi, jnp.int32(K))
            amin = jnp.min(cand, axis=0)
            xidx_v[jj] = jnp.full((L,), amin, jnp.int32)
            writes.append(
                pltpu.async_copy(xidx_v.at[jj], xidx_hbm.at[p], sem_out))
            writes.append(
                pltpu.async_copy(xbuf_v.at[jj], x_hbm.at[b, j], sem_out))
        for w in writes:
            w.wait()

    return k(scores, atn_idx, prev_x, lens)


def kernel(stims, embed, atn_idx, lens):
    idx = atn_idx.astype(jnp.int32)
    lens_flat = lens.reshape(PAIRS).astype(jnp.int32)
    x_part = _sc_rowdot(embed, idx, stims)
    scores = _tc_scores(stims, embed)
    x, xidx = _sc_merge(scores, idx, lens_flat, x_part)
    xIdx = xidx[:, 0].reshape(B, J)
    return (x, xIdx)


# speedup vs baseline: 1.6831x; 1.6831x over previous
import functools

import jax
import jax.numpy as jnp
from jax import lax
from jax.experimental import pallas as pl
from jax.experimental.pallas import tpu as pltpu
from jax.experimental.pallas import tpu_sc as plsc

B, J, K, H, V = 16, 4, 2048, 256, 65536
PAIRS = B * J
L = 16
NC, NS = 2, 16
NW = NC * NS
PPW = PAIRS // NW
VBLK = 4096
VS = 14 * VBLK
CH = 64
MAXC = K // CH


def _tc_scores(stims, embed):
    def mm(stims_ref, emb_ref, out_ref):
        out_ref[...] = lax.dot_general(
            stims_ref[...], emb_ref[...],
            dimension_numbers=(((1,), (1,)), ((), ())),
            preferred_element_type=jnp.float32,
            precision=lax.Precision.HIGHEST,
        )

    return pl.pallas_call(
        mm,
        grid=(VS // VBLK,),
        in_specs=[
            pl.BlockSpec((B, H), lambda i: (0, 0)),
            pl.BlockSpec((VBLK, H), lambda i: (i, 0)),
        ],
        out_specs=pl.BlockSpec((B, VBLK), lambda i: (0, i)),
        out_shape=jax.ShapeDtypeStruct((B, VS), jnp.float32),
    )(stims, embed)


def _sc_rowdot(embed, atn_idx, stims):
    mesh = plsc.VectorSubcoreMesh(core_axis_name="c", subcore_axis_name="s")

    @functools.partial(
        pl.kernel,
        mesh=mesh,
        compiler_params=pltpu.CompilerParams(needs_layout_passes=False),
        out_type=[jax.ShapeDtypeStruct((B, J, K), jnp.float32)],
        scratch_types=[
            pltpu.VMEM((PPW, K), jnp.int32),
            pltpu.VMEM((H,), jnp.float32),
            pltpu.VMEM((MAXC, CH), jnp.int32),
            pltpu.VMEM((K,), jnp.int32),
            pltpu.VMEM((PPW, K), jnp.float32),
            pltpu.VMEM((H,), jnp.float32),
            pltpu.VMEM((CH, H), jnp.float32),
            pltpu.VMEM((CH, H), jnp.float32),
            pltpu.SemaphoreType.DMA,
            pltpu.SemaphoreType.DMA,
            pltpu.SemaphoreType.DMA,
            pltpu.SemaphoreType.DMA,
        ],
    )
    def k(embed_hbm, idx_hbm, stims_hbm, x_hbm,
          idx_v, stims_v, cidx_v, cpos_v, xbuf_v, t_v, buf0, buf1,
          sem_in, semg0, semg1, sem_out):
        wid = lax.axis_index("s") * NC + lax.axis_index("c")
        b = wid // (NW // B)
        stage = [pltpu.async_copy(stims_hbm.at[b], stims_v, sem_in)]
        for jj in range(PPW):
            j = (wid * PPW + jj) % J
            stage.append(
                pltpu.async_copy(idx_hbm.at[b, j], idx_v.at[jj], sem_in))
        for c in stage:
            c.wait()

        s_list = [stims_v[pl.ds(h * L, L)] for h in range(H // L)]
        lane = lax.broadcasted_iota(jnp.int32, (L,), 0)
        zrow_i = jnp.zeros((L,), jnp.int32)

        def chunk_dot(cbase, buf, offv, jj):
            def group(g, carry):
                accs = [buf[g * L + r, pl.ds(0, L)] * s_list[0]
                        for r in range(L)]
                for h in range(1, H // L):
                    sh = s_list[h]
                    for r in range(L):
                        accs[r] = accs[r] + buf[g * L + r,
                                                pl.ds(h * L, L)] * sh
                for r in range(L):
                    t_v[pl.ds(r * L, L)] = accs[r]
                sums = plsc.load_gather(t_v, [lane * L])
                for l in range(1, L):
                    sums = sums + plsc.load_gather(t_v, [lane * L + l])
                ent = cbase * CH + g * L
                pos_v = cpos_v[pl.ds(ent, L)]
                valid = (ent + lane) < offv
                plsc.store_scatter(
                    xbuf_v, [jnp.full((L,), jj, jnp.int32), pos_v],
                    sums, mask=valid)
                return carry
            lax.fori_loop(0, CH // L, group, 0)

        def do_pair(jj, _):
            def zbody(i, c):
                cidx_v[i, pl.ds(0, L)] = zrow_i
                for q in range(1, CH // L):
                    cidx_v[i, pl.ds(q * L, L)] = zrow_i
                return c
            lax.fori_loop(0, MAXC, zbody, 0)

            def cbody(i, offv):
                idxv = idx_v[jj, pl.ds(i * L, L)]
                m = idxv >= VS
                pref = plsc.cumsum(m.astype(jnp.int32))
                pos = offv + pref - 1
                cnt = plsc.all_reduce_population_count(m)
                plsc.store_scatter(
                    cidx_v,
                    [jnp.right_shift(pos, 6), pos & (CH - 1)],
                    idxv, mask=m)
                plsc.store_scatter(cpos_v, [pos], lane + i * L, mask=m)
                return offv + cnt
            offv = lax.fori_loop(0, K // L, cbody,
                                 jnp.zeros((L,), jnp.int32))
            tot = jnp.max(offv, axis=0)
            nc = (tot + CH - 1) // CH
            nce = ((nc + 1) // 2) * 2

            @pl.when(nce > 0)
            def _():
                pltpu.async_copy(embed_hbm.at[cidx_v.at[0]], buf0, semg0)

            @pl.when(nce > 1)
            def _():
                pltpu.async_copy(embed_hbm.at[cidx_v.at[1]], buf1, semg1)

            def mbody(i, carry):
                c0 = 2 * i
                pltpu.make_async_copy(
                    embed_hbm.at[cidx_v.at[0]], buf0, semg0).wait()
                chunk_dot(c0, buf0, offv, jj)

                @pl.when(c0 + 2 < nce)
                def _():
                    pltpu.async_copy(
                        embed_hbm.at[cidx_v.at[c0 + 2]], buf0, semg0)

                pltpu.make_async_copy(
                    embed_hbm.at[cidx_v.at[1]], buf1, semg1).wait()
                chunk_dot(c0 + 1, buf1, offv, jj)

                @pl.when(c0 + 3 < nce)
                def _():
                    pltpu.async_copy(
                        embed_hbm.at[cidx_v.at[c0 + 3]], buf1, semg1)
                return carry
            lax.fori_loop(0, nce // 2, mbody, 0)
            return _

        for jj in range(PPW):
            do_pair(jj, None)
            j = (wid * PPW + jj) % J
            pltpu.async_copy(xbuf_v.at[jj], x_hbm.at[b, j], sem_out)
        for jj in range(PPW):
            j = (wid * PPW + jj) % J
            pltpu.make_async_copy(
                xbuf_v.at[jj], x_hbm.at[b, j], sem_out).wait()

    (x_part,) = k(embed, atn_idx, stims)
    return x_part


def _sc_merge(scores, atn_idx, lens, prev_x):
    mesh = plsc.VectorSubcoreMesh(core_axis_name="c", subcore_axis_name="s")

    @functools.partial(
        pl.kernel,
        mesh=mesh,
        compiler_params=pltpu.CompilerParams(needs_layout_passes=False),
        out_type=[
            jax.ShapeDtypeStruct((B, J, K), jnp.float32),
            jax.ShapeDtypeStruct((PAIRS, L), jnp.int32),
        ],
        scratch_types=[
            pltpu.VMEM((VS,), jnp.float32),
            pltpu.VMEM((PPW, K), jnp.int32),
            pltpu.VMEM((PPW, K), jnp.float32),
            pltpu.VMEM((PPW, K), jnp.float32),
            pltpu.SemaphoreType.DMA,
            pltpu.SemaphoreType.DMA,
            pltpu.VMEM((PPW, L), jnp.int32),
            pltpu.VMEM((PAIRS,), jnp.int32),
        ],
    )
    def k(scores_hbm, idx_hbm, prev_hbm, lens_hbm, x_hbm, xidx_hbm,
          row_v, idx_v, xbuf_v, prev_v, sem_in, sem_out, xidx_v, lens_v):
        wid = lax.axis_index("s") * NC + lax.axis_index("c")
        b = wid // (NW // B)
        copies = [pltpu.async_copy(scores_hbm.at[b], row_v, sem_in),
                  pltpu.async_copy(lens_hbm, lens_v, sem_in)]
        for jj in range(PPW):
            j = (wid * PPW + jj) % J
            copies.append(
                pltpu.async_copy(idx_hbm.at[b, j], idx_v.at[jj], sem_in))
            copies.append(
                pltpu.async_copy(prev_hbm.at[b, j], prev_v.at[jj], sem_in))
        for c in copies:
            c.wait()
        lane = lax.broadcasted_iota(jnp.int32, (L,), 0)
        neg = jnp.full((L,), -1e9, jnp.float32)
        zero = jnp.zeros((L,), jnp.int32)
        writes = []
        for jj in range(PPW):
            p = wid * PPW + jj
            j = p % J
            ln = plsc.load_gather(lens_v, [jnp.full((L,), p, jnp.int32)])

            def body(i, carry, jj=jj, ln=ln):
                best_val, best_idx = carry
                idxv = idx_v[jj, pl.ds(i * L, L)]
                inrange = idxv < VS
                safe = jnp.where(inrange, idxv, zero)
                vals = plsc.load_gather(row_v, [safe])
                prev = prev_v[jj, pl.ds(i * L, L)]
                merged = jnp.where(inrange, vals, prev)
                xbuf_v[jj, pl.ds(i * L, L)] = merged
                kv = lane + i * L
                mval = jnp.where(kv < ln, merged, neg)
                upd = mval > best_val
                return (jnp.where(upd, mval, best_val),
                        jnp.where(upd, kv, best_idx))

            bv0 = jnp.full((L,), -jnp.inf, jnp.float32)
            bi0 = jnp.zeros((L,), jnp.int32)
            bv, bi = lax.fori_loop(0, K // L, body, (bv0, bi0))
            mx = jnp.max(bv, axis=0)
            cand = jnp.where(bv == mx, bi, jnp.int32(K))
            amin = jnp.min(cand, axis=0)
            xidx_v[jj] = jnp.full((L,), amin, jnp.int32)
            writes.append(
                pltpu.async_copy(xidx_v.at[jj], xidx_hbm.at[p], sem_out))
            writes.append(
                pltpu.async_copy(xbuf_v.at[jj], x_hbm.at[b, j], sem_out))
        for w in writes:
            w.wait()

    return k(scores, atn_idx, prev_x, lens)


def kernel(stims, embed, atn_idx, lens):
    idx = atn_idx.astype(jnp.int32)
    lens_flat = lens.reshape(PAIRS).astype(jnp.int32)
    x_part = _sc_rowdot(embed, idx, stims)
    scores = _tc_scores(stims, embed)
    x, xidx = _sc_merge(scores, idx, lens_flat, x_part)
    xIdx = xidx[:, 0].reshape(B, J)
    return (x, xIdx)


# speedup vs baseline: 4.0797x; 2.4239x over previous
import functools

import jax
import jax.numpy as jnp
from jax import lax
from jax.experimental import pallas as pl
from jax.experimental.pallas import tpu as pltpu
from jax.experimental.pallas import tpu_sc as plsc

B, J, K, H, V = 16, 4, 2048, 256, 65536
PAIRS = B * J
L = 16
NC, NS = 2, 16
NW = NC * NS
PPW = PAIRS // NW
NPHASE = 2
VH = V // NPHASE
VBLK = 4096


def _tc_scores_phase(stims, embed, ph):
    def mm(stims_ref, emb_ref, out_ref):
        out_ref[...] = lax.dot_general(
            stims_ref[...], emb_ref[...],
            dimension_numbers=(((1,), (1,)), ((), ())),
            preferred_element_type=jnp.float32,
            precision=lax.Precision.HIGHEST,
        )

    base_blk = ph * (VH // VBLK)
    return pl.pallas_call(
        mm,
        grid=(VH // VBLK,),
        in_specs=[
            pl.BlockSpec((B, H), lambda i: (0, 0)),
            pl.BlockSpec((VBLK, H), lambda i: (base_blk + i, 0)),
        ],
        out_specs=pl.BlockSpec((B, VBLK), lambda i: (0, i)),
        out_shape=jax.ShapeDtypeStruct((B, VH), jnp.float32),
    )(stims, embed)


def _sc_phase(scores_ph, atn_idx, lens, prev_x, ph):
    last = ph == NPHASE - 1
    base = ph * VH
    mesh = plsc.VectorSubcoreMesh(core_axis_name="c", subcore_axis_name="s")

    out_type = [jax.ShapeDtypeStruct((B, J, K), jnp.float32)]
    scratch = [
        pltpu.VMEM((VH,), jnp.float32),
        pltpu.VMEM((PPW, K), jnp.int32),
        pltpu.VMEM((PPW, K), jnp.float32),
        pltpu.VMEM((PPW, K), jnp.float32),
        pltpu.SemaphoreType.DMA,
        pltpu.SemaphoreType.DMA,
    ]
    if last:
        out_type.append(jax.ShapeDtypeStruct((PAIRS, L), jnp.int32))
        scratch.append(pltpu.VMEM((PPW, L), jnp.int32))
        scratch.append(pltpu.VMEM((PAIRS,), jnp.int32))

    @functools.partial(
        pl.kernel,
        mesh=mesh,
        compiler_params=pltpu.CompilerParams(needs_layout_passes=False),
        out_type=out_type,
        scratch_types=scratch,
    )
    def k(scores_hbm, idx_hbm, *rest):
        prev_hbm = None
        if ph > 0:
            prev_hbm, *rest = rest
        if last:
            lens_hbm, x_hbm, xidx_hbm, row_v, idx_v, xbuf_v, prev_v, \
                sem_in, sem_out, xidx_v, lens_v = rest
        else:
            x_hbm, row_v, idx_v, xbuf_v, prev_v, sem_in, sem_out = rest
        wid = lax.axis_index("s") * NC + lax.axis_index("c")
        b = wid // (NW // B)
        copies = [pltpu.async_copy(scores_hbm.at[b], row_v, sem_in)]
        for jj in range(PPW):
            j = (wid * PPW + jj) % J
            copies.append(
                pltpu.async_copy(idx_hbm.at[b, j], idx_v.at[jj], sem_in))
            if ph > 0:
                copies.append(
                    pltpu.async_copy(prev_hbm.at[b, j], prev_v.at[jj],
                                     sem_in))
        if last:
            copies.append(pltpu.async_copy(lens_hbm, lens_v, sem_in))
        for c in copies:
            c.wait()
        lane = lax.broadcasted_iota(jnp.int32, (L,), 0)
        neg = jnp.full((L,), -1e9, jnp.float32)
        zero = jnp.zeros((L,), jnp.int32)
        writes = []
        for jj in range(PPW):
            p = wid * PPW + jj
            j = p % J
            if last:
                ln = plsc.load_gather(
                    lens_v, [jnp.full((L,), p, jnp.int32)])

            def body(i, carry, jj=jj):
                best_val, best_idx = carry
                idxv = idx_v[jj, pl.ds(i * L, L)]
                local = idxv - base
                inrange = (idxv >= base) & (local < VH)
                safe = jnp.where(inrange, local, zero)
                vals = plsc.load_gather(row_v, [safe])
                if ph > 0:
                    prev = prev_v[jj, pl.ds(i * L, L)]
                else:
                    prev = jnp.zeros((L,), jnp.float32)
                merged = jnp.where(inrange, vals, prev)
                xbuf_v[jj, pl.ds(i * L, L)] = merged
                if last:
                    kv = lane + i * L
                    mval = jnp.where(kv < ln, merged, neg)
                    upd = mval > best_val
                    return (jnp.where(upd, mval, best_val),
                            jnp.where(upd, kv, best_idx))
                return carry

            bv0 = jnp.full((L,), -jnp.inf, jnp.float32)
            bi0 = jnp.zeros((L,), jnp.int32)
            bv, bi = lax.fori_loop(0, K // L, body, (bv0, bi0))
            if last:
                mx = jnp.max(bv, axis=0)
                cand = jnp.where(bv == mx, bi, jnp.int32(K))
                amin = jnp.min(cand, axis=0)
                xidx_v[jj] = jnp.full((L,), amin, jnp.int32)
                writes.append(
                    pltpu.async_copy(xidx_v.at[jj], xidx_hbm.at[p],
                                     sem_out))
            writes.append(
                pltpu.async_copy(xbuf_v.at[jj], x_hbm.at[b, j], sem_out))
        for w in writes:
            w.wait()

    args = [scores_ph, atn_idx]
    if ph > 0:
        args.append(prev_x)
    if last:
        args.append(lens)
    return k(*args)


def kernel(stims, embed, atn_idx, lens):
    idx = atn_idx.astype(jnp.int32)
    lens_flat = lens.reshape(PAIRS).astype(jnp.int32)
    x = None
    for ph in range(NPHASE - 1):
        scores_ph = _tc_scores_phase(stims, embed, ph)
        (x,) = _sc_phase(scores_ph, idx, lens_flat, x, ph)
    scores_ph = _tc_scores_phase(stims, embed, NPHASE - 1)
    x, xidx = _sc_phase(scores_ph, idx, lens_flat, x, NPHASE - 1)
    xIdx = xidx[:, 0].reshape(B, J)
    return (x, xIdx)
